# baseline (speedup 1.0000x reference)
import functools

import jax
import jax.numpy as jnp
from jax import lax
from jax.experimental import pallas as pl
from jax.experimental.pallas import tpu as pltpu
from jax.experimental.pallas import tpu_sc as plsc

_EPS = 1e-5
_G = 500
_GP = 512
_NC = 2
_NS = 16
_C = 128
_ZR = 80


def _mesh():
    return plsc.VectorSubcoreMesh(core_axis_name="c", subcore_axis_name="s")


def _sc_degree(dst3, n):
    nw, p_chunks, c = dst3.shape
    row_chunks = n // _ZR

    @functools.partial(
        pl.kernel,
        out_type=jax.ShapeDtypeStruct((_NC, n, 16), jnp.float32),
        mesh=_mesh(),
        scratch_types=[
            pltpu.VMEM((p_chunks, c), jnp.int32),
            pltpu.VMEM((1, c), jnp.int32),
            pltpu.VMEM((c, 16), jnp.float32),
            pltpu.VMEM((_ZR, 16), jnp.float32),
            pltpu.VMEM_SHARED((n + 8, 16), jnp.float32),
        ],
    )
    def k(dst_hbm, out_hbm, didx, dsm, ones_v, ztile, acc):
        cid = lax.axis_index("c")
        sid = lax.axis_index("s")
        wid = cid * _NS + sid

        one16 = jnp.ones((16,), jnp.float32)
        zero16 = jnp.zeros((16,), jnp.float32)

        @pl.loop(0, c)
        def _(i):
            ones_v[i] = one16

        @pl.loop(0, _ZR)
        def _(i):
            ztile[i] = zero16

        @pl.loop(sid, row_chunks, step=_NS)
        def _(t):
            pltpu.sync_copy(ztile, acc.at[pl.ds(t * _ZR, _ZR)])

        pltpu.sync_copy(dst_hbm.at[wid], didx)

        plsc.subcore_barrier()

        @pl.loop(0, p_chunks)
        def _(t):
            pltpu.sync_copy(ones_v, acc.at[didx.at[t]], add=True)

        plsc.subcore_barrier()

        @pl.loop(sid, row_chunks, step=_NS)
        def _(t):
            pltpu.sync_copy(acc.at[pl.ds(t * _ZR, _ZR)],
                            out_hbm.at[cid].at[pl.ds(t * _ZR, _ZR)])

    return k(dst3)


_NBUF = 2
_HALVES = 2


def _sc_scatter(u, src3, dst3, n):
    d = u.shape[1]
    nw, p_chunks, c = src3.shape
    hc = p_chunks // _HALVES
    row_chunks = n // _ZR

    @functools.partial(
        pl.kernel,
        out_type=jax.ShapeDtypeStruct((_NC, n, d), jnp.float32),
        mesh=_mesh(),
        scratch_types=[
            pltpu.VMEM((hc, c), jnp.int32),
            pltpu.VMEM((hc, c), jnp.int32),
            pltpu.VMEM((_NBUF, c, d), jnp.float32),
            pltpu.VMEM((40, d), jnp.float32),
            pltpu.VMEM_SHARED((n, d), jnp.float32),
            pltpu.SemaphoreType.DMA,
            pltpu.SemaphoreType.DMA,
        ],
    )
    def k(u_hbm, src_hbm, dst_hbm, out_hbm, sidx, didx, rows, ztile,
          acc, *sems):
        cid = lax.axis_index("c")
        sid = lax.axis_index("s")
        wid = cid * _NS + sid

        zero16 = jnp.zeros((16,), jnp.float32)

        @pl.loop(0, 40)
        def _(i):
            @pl.loop(0, d, step=16)
            def _(j):
                ztile[i, pl.ds(j, 16)] = zero16

        @pl.loop(sid, n // 40, step=_NS)
        def _(t):
            pltpu.sync_copy(ztile, acc.at[pl.ds(t * 40, 40)])

        plsc.subcore_barrier()

        def gather(kk, j):
            return pltpu.make_async_copy(u_hbm.at[sidx.at[kk]], rows.at[j],
                                         sems[j])

        for half in range(_HALVES):
            pltpu.sync_copy(src_hbm.at[wid].at[pl.ds(half * hc, hc)], sidx)
            pltpu.sync_copy(dst_hbm.at[wid].at[pl.ds(half * hc, hc)], didx)

            for j in range(_NBUF):
                gather(j, j).start()

            @pl.loop(0, hc // _NBUF)
            def _(q):
                for j in range(_NBUF):
                    kk = q * _NBUF + j
                    gather(kk, j).wait()
                    pltpu.sync_copy(rows.at[j], acc.at[didx.at[kk]],
                                    add=True)

                    @pl.when(kk + _NBUF < hc)
                    def _():
                        gather(kk + _NBUF, j).start()

        plsc.subcore_barrier()

        @pl.loop(sid, row_chunks, step=_NS)
        def _(t):
            pltpu.sync_copy(acc.at[pl.ds(t * _ZR, _ZR)],
                            out_hbm.at[cid].at[pl.ds(t * _ZR, _ZR)])

    return k(u, src3, dst3)


def _sc_pool(h, batch):
    n, d = h.shape
    row_chunks = n // _ZR
    g_chunks = 8
    gr = _GP // g_chunks

    @functools.partial(
        pl.kernel,
        out_type=(jax.ShapeDtypeStruct((_NC, _GP, d), jnp.float32),
                  jax.ShapeDtypeStruct((_NC, _GP, 16), jnp.float32)),
        mesh=_mesh(),
        scratch_types=[
            pltpu.VMEM((1, _ZR), jnp.int32),
            pltpu.VMEM((_ZR, d), jnp.float32),
            pltpu.VMEM((_ZR, 16), jnp.float32),
            pltpu.VMEM((gr, d), jnp.float32),
            pltpu.VMEM((gr, 16), jnp.float32),
            pltpu.VMEM_SHARED((_GP, d), jnp.float32),
            pltpu.VMEM_SHARED((_GP, 16), jnp.float32),
        ],
    )
    def k(h_hbm, b_hbm, osum_hbm, ocnt_hbm,
          bidx, hrows, ones_v, zs, zc, acc_s, acc_c):
        cid = lax.axis_index("c")
        sid = lax.axis_index("s")
        wid = cid * _NS + sid

        one16 = jnp.ones((16,), jnp.float32)
        zero16 = jnp.zeros((16,), jnp.float32)

        @pl.loop(0, _ZR)
        def _(i):
            ones_v[i] = one16

        @pl.loop(0, gr)
        def _(i):
            zc[i] = zero16

            @pl.loop(0, d, step=16)
            def _(j):
                zs[i, pl.ds(j, 16)] = zero16

        @pl.when(sid < g_chunks)
        def _():
            pltpu.sync_copy(zs, acc_s.at[pl.ds(sid * gr, gr)])
            pltpu.sync_copy(zc, acc_c.at[pl.ds(sid * gr, gr)])

        plsc.subcore_barrier()

        @pl.loop(wid, row_chunks, step=_NC * _NS)
        def _(t):
            pltpu.sync_copy(b_hbm.at[pl.ds(t * _ZR, _ZR)], bidx.at[0])
            pltpu.sync_copy(h_hbm.at[pl.ds(t * _ZR, _ZR)], hrows)
            pltpu.sync_copy(hrows, acc_s.at[bidx.at[0]], add=True)
            pltpu.sync_copy(ones_v, acc_c.at[bidx.at[0]], add=True)

        plsc.subcore_barrier()

        @pl.when(sid < g_chunks)
        def _():
            pltpu.sync_copy(acc_s.at[pl.ds(sid * gr, gr)],
                            osum_hbm.at[cid].at[pl.ds(sid * gr, gr)])
            pltpu.sync_copy(acc_c.at[pl.ds(sid * gr, gr)],
                            ocnt_hbm.at[cid].at[pl.ds(sid * gr, gr)])

    return k(h, batch)


_BLK = 1000


def _dot(a, b):
    return jnp.dot(a, b, preferred_element_type=jnp.float32,
                   precision=lax.Precision.HIGHEST)


def _tc_embed(x, emb_W, emb_b, cw0):
    n, d = x.shape

    def body(x_ref, w_ref, b_ref, cw_ref, o_ref):
        h = _dot(x_ref[...], w_ref[...]) + b_ref[...]
        o_ref[...] = _dot(h, cw_ref[...])

    return pl.pallas_call(
        body,
        grid=(n // _BLK,),
        in_specs=[
            pl.BlockSpec((_BLK, d), lambda i: (i, 0)),
            pl.BlockSpec((d, d), lambda i: (0, 0)),
            pl.BlockSpec((1, d), lambda i: (0, 0)),
            pl.BlockSpec((d, d), lambda i: (0, 0)),
        ],
        out_specs=pl.BlockSpec((_BLK, d), lambda i: (i, 0)),
        out_shape=jax.ShapeDtypeStruct((n, d), jnp.float32),
    )(x, emb_W, emb_b.reshape(1, d), cw0)


def _dinv_from(degp_ref):
    deg = degp_ref[0, :, 0] + degp_ref[1, :, 0] + 1.0
    return lax.rsqrt(deg)[:, None]


def _tc_scale(m0, degp):
    n, d = m0.shape

    def body(m_ref, g_ref, o_ref):
        o_ref[...] = _dinv_from(g_ref) * m_ref[...]

    return pl.pallas_call(
        body,
        grid=(n // _BLK,),
        in_specs=[
            pl.BlockSpec((_BLK, d), lambda i: (i, 0)),
            pl.BlockSpec((2, _BLK, 16), lambda i: (0, i, 0)),
        ],
        out_specs=pl.BlockSpec((_BLK, d), lambda i: (i, 0)),
        out_shape=jax.ShapeDtypeStruct((n, d), jnp.float32),
    )(m0, degp)


def _bn_relu(agg, g_ref, b_ref):
    scale = g_ref[...] * (1.0 / jnp.sqrt(1.0 + _EPS))
    return jnp.maximum(scale * agg + b_ref[...], 0.0)


def _tc_layer(p, u, degp, bn_g, bn_b, conv_b, cw_next):
    n, d = u.shape

    def body(p_ref, u_ref, g_ref, bg_ref, bb_ref, cb_ref, cw_ref, o_ref):
        dinv = _dinv_from(g_ref)
        agg = dinv * (p_ref[0] + p_ref[1] + u_ref[...]) + cb_ref[...]
        h = _bn_relu(agg, bg_ref, bb_ref)
        o_ref[...] = dinv * _dot(h, cw_ref[...])

    return pl.pallas_call(
        body,
        grid=(n // _BLK,),
        in_specs=[
            pl.BlockSpec((2, _BLK, d), lambda i: (0, i, 0)),
            pl.BlockSpec((_BLK, d), lambda i: (i, 0)),
            pl.BlockSpec((2, _BLK, 16), lambda i: (0, i, 0)),
            pl.BlockSpec((1, d), lambda i: (0, 0)),
            pl.BlockSpec((1, d), lambda i: (0, 0)),
            pl.BlockSpec((1, d), lambda i: (0, 0)),
            pl.BlockSpec((d, d), lambda i: (0, 0)),
        ],
        out_specs=pl.BlockSpec((_BLK, d), lambda i: (i, 0)),
        out_shape=jax.ShapeDtypeStruct((n, d), jnp.float32),
    )(p, u, degp, bn_g.reshape(1, d), bn_b.reshape(1, d),
      conv_b.reshape(1, d), cw_next)


def _tc_last(p, u, degp, bn_g, bn_b, conv_b):
    n, d = u.shape

    def body(p_ref, u_ref, g_ref, bg_ref, bb_ref, cb_ref, o_ref):
        dinv = _dinv_from(g_ref)
        agg = dinv * (p_ref[0] + p_ref[1] + u_ref[...]) + cb_ref[...]
        o_ref[...] = _bn_relu(agg, bg_ref, bb_ref)

    return pl.pallas_call(
        body,
        grid=(n // _BLK,),
        in_specs=[
            pl.BlockSpec((2, _BLK, d), lambda i: (0, i, 0)),
            pl.BlockSpec((_BLK, d), lambda i: (i, 0)),
            pl.BlockSpec((2, _BLK, 16), lambda i: (0, i, 0)),
            pl.BlockSpec((1, d), lambda i: (0, 0)),
            pl.BlockSpec((1, d), lambda i: (0, 0)),
            pl.BlockSpec((1, d), lambda i: (0, 0)),
        ],
        out_specs=pl.BlockSpec((_BLK, d), lambda i: (i, 0)),
        out_shape=jax.ShapeDtypeStruct((n, d), jnp.float32),
    )(p, u, degp, bn_g.reshape(1, d), bn_b.reshape(1, d), conv_b.reshape(1, d))


def _tc_head(sums, cnts, mlp_W1, mlp_b1, mlp_bn_g, mlp_bn_b, mlp_W2, mlp_b2):
    g, d = sums.shape[1], sums.shape[2]

    def body(s_ref, c_ref, w1_ref, b1_ref, g_ref, b_ref, w2_ref, b2_ref,
             o_ref):
        cnt = c_ref[0, :, 0] + c_ref[1, :, 0]
        pooled = (s_ref[0] + s_ref[1]) / jnp.maximum(cnt, 1.0)[:, None]
        t = _dot(pooled, w1_ref[...]) + b1_ref[...]
        h2 = _bn_relu(t, g_ref, b_ref)
        o_ref[...] = jnp.sum(h2 * w2_ref[...], axis=1,
                             keepdims=True) + b2_ref[...]

    out = pl.pallas_call(
        body,
        out_shape=jax.ShapeDtypeStruct((g, 1), jnp.float32),
    )(sums, cnts, mlp_W1, mlp_b1.reshape(1, d), mlp_bn_g.reshape(1, d),
      mlp_bn_b.reshape(1, d), mlp_W2.reshape(1, d), mlp_b2.reshape(1, 1))
    return out[:_G]


def kernel(x, edge_index, batch, emb_W, emb_b, conv_W, conv_b, bn_g, bn_b,
           mlp_W1, mlp_b1, mlp_bn_g, mlp_bn_b, mlp_W2, mlp_b2):
    n, d = x.shape
    num_layers = conv_W.shape[0]
    src = edge_index[0]
    dst = edge_index[1]

    e = src.shape[0]
    nw = _NC * _NS
    p_chunks = 80
    e_pad = nw * p_chunks * _C
    npad = e_pad - e
    src3 = jnp.concatenate(
        [src, jnp.full((npad,), n, src.dtype)]).reshape(nw, p_chunks, _C)
    dst3 = jnp.concatenate(
        [dst, (jnp.arange(npad, dtype=dst.dtype) * 37) % n],
    ).reshape(nw, p_chunks, _C)
    dst3d = jnp.concatenate(
        [dst, jnp.full((npad,), n, dst.dtype)]).reshape(nw, p_chunks, _C)
    zrows = jnp.zeros((8, d), jnp.float32)

    degp = _sc_degree(dst3d, n)
    m0 = _tc_embed(x, emb_W, emb_b, conv_W[0])
    u = _tc_scale(m0, degp)
    h = None
    for l in range(num_layers):
        p = _sc_scatter(jnp.concatenate([u, zrows]), src3, dst3, n)
        if l + 1 < num_layers:
            u = _tc_layer(p, u, degp, bn_g[l], bn_b[l], conv_b[l],
                          conv_W[l + 1])
        else:
            h = _tc_last(p, u, degp, bn_g[l], bn_b[l], conv_b[l])
    sums, cnts = _sc_pool(h, batch)
    return _tc_head(sums, cnts, mlp_W1, mlp_b1, mlp_bn_g, mlp_bn_b,
                    mlp_W2, mlp_b2)


# speedup vs baseline: 3.1763x; 3.1763x over previous
import functools

import jax
import jax.numpy as jnp
import numpy as np
from jax import lax
from jax.experimental import pallas as pl
from jax.experimental.pallas import tpu as pltpu
from jax.experimental.pallas import tpu_sc as plsc

_EPS = 1e-5
_G = 500
_GP = 512
_NC = 2
_NS = 16
_C = 128
_ZR = 80


def _mesh():
    return plsc.VectorSubcoreMesh(core_axis_name="c", subcore_axis_name="s")


def _sc_degree(dst3, n):
    nw, p_chunks, c = dst3.shape
    row_chunks = n // _ZR

    @functools.partial(
        pl.kernel,
        out_type=jax.ShapeDtypeStruct((_NC, n, 16), jnp.float32),
        mesh=_mesh(),
        scratch_types=[
            pltpu.VMEM((p_chunks, c), jnp.int32),
            pltpu.VMEM((1, c), jnp.int32),
            pltpu.VMEM((c, 16), jnp.float32),
            pltpu.VMEM((_ZR, 16), jnp.float32),
            pltpu.VMEM_SHARED((n + 8, 16), jnp.float32),
        ],
    )
    def k(dst_hbm, out_hbm, didx, dsm, ones_v, ztile, acc):
        cid = lax.axis_index("c")
        sid = lax.axis_index("s")
        wid = cid * _NS + sid

        one16 = jnp.ones((16,), jnp.float32)
        zero16 = jnp.zeros((16,), jnp.float32)

        @pl.loop(0, c)
        def _(i):
            ones_v[i] = one16

        @pl.loop(0, _ZR)
        def _(i):
            ztile[i] = zero16

        @pl.loop(sid, row_chunks, step=_NS)
        def _(t):
            pltpu.sync_copy(ztile, acc.at[pl.ds(t * _ZR, _ZR)])

        pltpu.sync_copy(dst_hbm.at[wid], didx)

        plsc.subcore_barrier()

        @pl.loop(0, p_chunks)
        def _(t):
            pltpu.sync_copy(ones_v, acc.at[didx.at[t]], add=True)

        plsc.subcore_barrier()

        @pl.loop(sid, row_chunks, step=_NS)
        def _(t):
            pltpu.sync_copy(acc.at[pl.ds(t * _ZR, _ZR)],
                            out_hbm.at[cid].at[pl.ds(t * _ZR, _ZR)])

    return k(dst3)


_NBUF = 2
_HALVES = 2


def _sc_scatter(u, src3, dst3):
    n, d = u.shape
    nw, p_chunks, c = src3.shape
    hc = p_chunks // _HALVES
    row_chunks = n // _ZR

    @functools.partial(
        pl.kernel,
        out_type=jax.ShapeDtypeStruct((_NC, n, d), jnp.float32),
        mesh=_mesh(),
        scratch_types=[
            pltpu.VMEM((hc, c), jnp.int32),
            pltpu.VMEM((hc, c), jnp.int32),
            pltpu.VMEM((_NBUF, c, d), jnp.float32),
            pltpu.VMEM((40, d), jnp.float32),
            pltpu.VMEM_SHARED((n, d), jnp.float32),
            pltpu.SemaphoreType.DMA,
            pltpu.SemaphoreType.DMA,
        ],
    )
    def k(u_hbm, src_hbm, dst_hbm, out_hbm, sidx, didx, rows, ztile,
          acc, *sems):
        cid = lax.axis_index("c")
        sid = lax.axis_index("s")
        wid = cid * _NS + sid

        zero16 = jnp.zeros((16,), jnp.float32)

        @pl.loop(0, 40)
        def _(i):
            @pl.loop(0, d, step=16)
            def _(j):
                ztile[i, pl.ds(j, 16)] = zero16

        @pl.loop(sid, n // 40, step=_NS)
        def _(t):
            pltpu.sync_copy(ztile, acc.at[pl.ds(t * 40, 40)])

        plsc.subcore_barrier()

        def gather(kk, j):
            return pltpu.make_async_copy(u_hbm.at[sidx.at[kk]], rows.at[j],
                                         sems[j])

        for half in range(_HALVES):
            pltpu.sync_copy(src_hbm.at[wid].at[pl.ds(half * hc, hc)], sidx)
            pltpu.sync_copy(dst_hbm.at[wid].at[pl.ds(half * hc, hc)], didx)

            for j in range(_NBUF):
                gather(j, j).start()

            @pl.loop(0, hc // _NBUF)
            def _(q):
                for j in range(_NBUF):
                    kk = q * _NBUF + j
                    gather(kk, j).wait()
                    pltpu.sync_copy(rows.at[j], acc.at[didx.at[kk]],
                                    add=True)

                    @pl.when(kk + _NBUF < hc)
                    def _():
                        gather(kk + _NBUF, j).start()

        plsc.subcore_barrier()

        @pl.loop(sid, row_chunks, step=_NS)
        def _(t):
            pltpu.sync_copy(acc.at[pl.ds(t * _ZR, _ZR)],
                            out_hbm.at[cid].at[pl.ds(t * _ZR, _ZR)])

    return k(u, src3, dst3)


def _sc_pool(h, batch):
    n, d = h.shape
    row_chunks = n // _ZR
    g_chunks = 8
    gr = _GP // g_chunks

    @functools.partial(
        pl.kernel,
        out_type=(jax.ShapeDtypeStruct((_NC, _GP, d), jnp.float32),
                  jax.ShapeDtypeStruct((_NC, _GP, 16), jnp.float32)),
        mesh=_mesh(),
        scratch_types=[
            pltpu.VMEM((1, _ZR), jnp.int32),
            pltpu.VMEM((_ZR, d), jnp.float32),
            pltpu.VMEM((_ZR, 16), jnp.float32),
            pltpu.VMEM((gr, d), jnp.float32),
            pltpu.VMEM((gr, 16), jnp.float32),
            pltpu.VMEM_SHARED((_GP, d), jnp.float32),
            pltpu.VMEM_SHARED((_GP, 16), jnp.float32),
        ],
    )
    def k(h_hbm, b_hbm, osum_hbm, ocnt_hbm,
          bidx, hrows, ones_v, zs, zc, acc_s, acc_c):
        cid = lax.axis_index("c")
        sid = lax.axis_index("s")
        wid = cid * _NS + sid

        one16 = jnp.ones((16,), jnp.float32)
        zero16 = jnp.zeros((16,), jnp.float32)

        @pl.loop(0, _ZR)
        def _(i):
            ones_v[i] = one16

        @pl.loop(0, gr)
        def _(i):
            zc[i] = zero16

            @pl.loop(0, d, step=16)
            def _(j):
                zs[i, pl.ds(j, 16)] = zero16

        @pl.when(sid < g_chunks)
        def _():
            pltpu.sync_copy(zs, acc_s.at[pl.ds(sid * gr, gr)])
            pltpu.sync_copy(zc, acc_c.at[pl.ds(sid * gr, gr)])

        plsc.subcore_barrier()

        @pl.loop(wid, row_chunks, step=_NC * _NS)
        def _(t):
            pltpu.sync_copy(b_hbm.at[pl.ds(t * _ZR, _ZR)], bidx.at[0])
            pltpu.sync_copy(h_hbm.at[pl.ds(t * _ZR, _ZR)], hrows)
            pltpu.sync_copy(hrows, acc_s.at[bidx.at[0]], add=True)
            pltpu.sync_copy(ones_v, acc_c.at[bidx.at[0]], add=True)

        plsc.subcore_barrier()

        @pl.when(sid < g_chunks)
        def _():
            pltpu.sync_copy(acc_s.at[pl.ds(sid * gr, gr)],
                            osum_hbm.at[cid].at[pl.ds(sid * gr, gr)])
            pltpu.sync_copy(acc_c.at[pl.ds(sid * gr, gr)],
                            ocnt_hbm.at[cid].at[pl.ds(sid * gr, gr)])

    return k(h, batch)


_BLK = 1000


def _dot(a, b):
    return jnp.dot(a, b, preferred_element_type=jnp.float32,
                   precision=lax.Precision.HIGHEST)


def _tc_embed(x, emb_W, emb_b, cw0):
    n, d = x.shape

    def body(x_ref, w_ref, b_ref, cw_ref, o_ref):
        h = _dot(x_ref[...], w_ref[...]) + b_ref[...]
        o_ref[...] = _dot(h, cw_ref[...])

    return pl.pallas_call(
        body,
        grid=(n // _BLK,),
        in_specs=[
            pl.BlockSpec((_BLK, d), lambda i: (i, 0)),
            pl.BlockSpec((d, d), lambda i: (0, 0)),
            pl.BlockSpec((1, d), lambda i: (0, 0)),
            pl.BlockSpec((d, d), lambda i: (0, 0)),
        ],
        out_specs=pl.BlockSpec((_BLK, d), lambda i: (i, 0)),
        out_shape=jax.ShapeDtypeStruct((n, d), jnp.float32),
    )(x, emb_W, emb_b.reshape(1, d), cw0)


def _dinv_from(degp_ref):
    deg = degp_ref[0, :, 0] + degp_ref[1, :, 0] + 1.0
    return lax.rsqrt(deg)[:, None]


def _tc_scale(m0, degp):
    n, d = m0.shape

    def body(m_ref, g_ref, o_ref):
        o_ref[...] = _dinv_from(g_ref) * m_ref[...]

    return pl.pallas_call(
        body,
        grid=(n // _BLK,),
        in_specs=[
            pl.BlockSpec((_BLK, d), lambda i: (i, 0)),
            pl.BlockSpec((2, _BLK, 16), lambda i: (0, i, 0)),
        ],
        out_specs=pl.BlockSpec((_BLK, d), lambda i: (i, 0)),
        out_shape=jax.ShapeDtypeStruct((n, d), jnp.float32),
    )(m0, degp)


def _bn_relu(agg, g_ref, b_ref):
    scale = g_ref[...] * (1.0 / jnp.sqrt(1.0 + _EPS))
    return jnp.maximum(scale * agg + b_ref[...], 0.0)


def _tc_layer(p, u, selfc, degp, bn_g, bn_b, conv_b, cw_next):
    n, d = u.shape

    def body(p_ref, u_ref, sc_ref, g_ref, bg_ref, bb_ref, cb_ref, cw_ref,
             o_ref):
        dinv = _dinv_from(g_ref)
        agg = dinv * (p_ref[0] + p_ref[1] + sc_ref[...] * u_ref[...]) \
            + cb_ref[...]
        h = _bn_relu(agg, bg_ref, bb_ref)
        o_ref[...] = dinv * _dot(h, cw_ref[...])

    return pl.pallas_call(
        body,
        grid=(n // _BLK,),
        in_specs=[
            pl.BlockSpec((2, _BLK, d), lambda i: (0, i, 0)),
            pl.BlockSpec((_BLK, d), lambda i: (i, 0)),
            pl.BlockSpec((_BLK, 1), lambda i: (i, 0)),
            pl.BlockSpec((2, _BLK, 16), lambda i: (0, i, 0)),
            pl.BlockSpec((1, d), lambda i: (0, 0)),
            pl.BlockSpec((1, d), lambda i: (0, 0)),
            pl.BlockSpec((1, d), lambda i: (0, 0)),
            pl.BlockSpec((d, d), lambda i: (0, 0)),
        ],
        out_specs=pl.BlockSpec((_BLK, d), lambda i: (i, 0)),
        out_shape=jax.ShapeDtypeStruct((n, d), jnp.float32),
    )(p, u, selfc, degp, bn_g.reshape(1, d), bn_b.reshape(1, d),
      conv_b.reshape(1, d), cw_next)


def _tc_last(p, u, selfc, degp, bn_g, bn_b, conv_b):
    n, d = u.shape

    def body(p_ref, u_ref, sc_ref, g_ref, bg_ref, bb_ref, cb_ref, o_ref):
        dinv = _dinv_from(g_ref)
        agg = dinv * (p_ref[0] + p_ref[1] + sc_ref[...] * u_ref[...]) \
            + cb_ref[...]
        o_ref[...] = _bn_relu(agg, bg_ref, bb_ref)

    return pl.pallas_call(
        body,
        grid=(n // _BLK,),
        in_specs=[
            pl.BlockSpec((2, _BLK, d), lambda i: (0, i, 0)),
            pl.BlockSpec((_BLK, d), lambda i: (i, 0)),
            pl.BlockSpec((_BLK, 1), lambda i: (i, 0)),
            pl.BlockSpec((2, _BLK, 16), lambda i: (0, i, 0)),
            pl.BlockSpec((1, d), lambda i: (0, 0)),
            pl.BlockSpec((1, d), lambda i: (0, 0)),
            pl.BlockSpec((1, d), lambda i: (0, 0)),
        ],
        out_specs=pl.BlockSpec((_BLK, d), lambda i: (i, 0)),
        out_shape=jax.ShapeDtypeStruct((n, d), jnp.float32),
    )(p, u, selfc, degp, bn_g.reshape(1, d), bn_b.reshape(1, d),
      conv_b.reshape(1, d))


def _tc_head(sums, cnts, mlp_W1, mlp_b1, mlp_bn_g, mlp_bn_b, mlp_W2, mlp_b2):
    g, d = sums.shape[1], sums.shape[2]

    def body(s_ref, c_ref, w1_ref, b1_ref, g_ref, b_ref, w2_ref, b2_ref,
             o_ref):
        cnt = c_ref[0, :, 0] + c_ref[1, :, 0]
        pooled = (s_ref[0] + s_ref[1]) / jnp.maximum(cnt, 1.0)[:, None]
        t = _dot(pooled, w1_ref[...]) + b1_ref[...]
        h2 = _bn_relu(t, g_ref, b_ref)
        o_ref[...] = jnp.sum(h2 * w2_ref[...], axis=1,
                             keepdims=True) + b2_ref[...]

    out = pl.pallas_call(
        body,
        out_shape=jax.ShapeDtypeStruct((g, 1), jnp.float32),
    )(sums, cnts, mlp_W1, mlp_b1.reshape(1, d), mlp_bn_g.reshape(1, d),
      mlp_bn_b.reshape(1, d), mlp_W2.reshape(1, d), mlp_b2.reshape(1, 1))
    return out[:_G]


def kernel(x, edge_index, batch, emb_W, emb_b, conv_W, conv_b, bn_g, bn_b,
           mlp_W1, mlp_b1, mlp_bn_g, mlp_bn_b, mlp_W2, mlp_b2):
    n, d = x.shape
    num_layers = conv_W.shape[0]
    src = edge_index[0]
    dst = edge_index[1]

    e = src.shape[0]
    nw = _NC * _NS
    p_chunks = 80
    e_pad = nw * p_chunks * _C
    npad = e_pad - e
    padr_np = (np.arange(npad, dtype=np.int64) * 37) % n
    padr = jnp.asarray(padr_np, dst.dtype)
    src3 = jnp.concatenate([src, padr]).reshape(nw, p_chunks, _C)
    dst3 = jnp.concatenate([dst, padr]).reshape(nw, p_chunks, _C)
    dst3d = jnp.concatenate(
        [dst, jnp.full((npad,), n, dst.dtype)]).reshape(nw, p_chunks, _C)
    selfc = jnp.asarray(
        1.0 - np.bincount(padr_np, minlength=n), jnp.float32).reshape(n, 1)

    degp = _sc_degree(dst3d, n)
    m0 = _tc_embed(x, emb_W, emb_b, conv_W[0])
    u = _tc_scale(m0, degp)
    h = None
    for l in range(num_layers):
        p = _sc_scatter(u, src3, dst3)
        if l + 1 < num_layers:
            u = _tc_layer(p, u, selfc, degp, bn_g[l], bn_b[l], conv_b[l],
                          conv_W[l + 1])
        else:
            h = _tc_last(p, u, selfc, degp, bn_g[l], bn_b[l], conv_b[l])
    sums, cnts = _sc_pool(h, batch)
    return _tc_head(sums, cnts, mlp_W1, mlp_b1, mlp_bn_g, mlp_bn_b,
                    mlp_W2, mlp_b2)
